# baseline (speedup 1.0000x reference)
import functools

import jax
import jax.numpy as jnp
from jax import lax
from jax.experimental import pallas as pl
from jax.experimental.pallas import tpu as pltpu
from jax.experimental.pallas import tpu_sc as plsc

_BATCH = 16384
_NW = 32
_CHUNK = _BATCH // _NW
_L = 16
_MAX_CHARGE = 6


def _sc_body(mz_hbm, ch_hbm, sl_hbm, ic_hbm, out_hbm, mz_v, ch_v, out_v,
             sl_v, ic_v):
    wid = lax.axis_index("s") * 2 + lax.axis_index("c")
    base = wid * _CHUNK
    pltpu.sync_copy(mz_hbm.at[pl.ds(base, _CHUNK)], mz_v)
    pltpu.sync_copy(ch_hbm.at[pl.ds(base, _CHUNK)], ch_v)
    pltpu.sync_copy(sl_hbm, sl_v)
    pltpu.sync_copy(ic_hbm, ic_v)

    def step(i, _):
        sl_ix = pl.ds(i * _L, _L)
        x = mz_v[sl_ix]
        c = ch_v[sl_ix]
        c = jnp.minimum(jnp.maximum(c, 0), _MAX_CHARGE)
        slope = plsc.load_gather(sl_v, [c])
        inter = plsc.load_gather(ic_v, [c])
        h = x * jnp.float32(0.5)
        yi = jnp.int32(0x5F3759DF) - (plsc.bitcast(x, jnp.int32) >> 1)
        y = plsc.bitcast(yi, jnp.float32)
        y = y * (jnp.float32(1.5) - h * y * y)
        y = y * (jnp.float32(1.5) - h * y * y)
        y = y * (jnp.float32(1.5) - h * y * y)
        s = x * y
        out_v[sl_ix] = slope * s + inter
        return 0

    lax.fori_loop(0, _CHUNK // _L, step, 0, unroll=4)
    pltpu.sync_copy(out_v, out_hbm.at[pl.ds(base, _CHUNK)])


@functools.partial(jax.jit, static_argnames=())
def _sc_call(mz, charge_i32, slopes16, intercepts16):
    mesh = plsc.VectorSubcoreMesh(core_axis_name="c", subcore_axis_name="s")
    return pl.kernel(
        _sc_body,
        out_type=jax.ShapeDtypeStruct((_BATCH,), jnp.float32),
        mesh=mesh,
        compiler_params=pltpu.CompilerParams(needs_layout_passes=False),
        scratch_types=[
            pltpu.VMEM((_CHUNK,), jnp.float32),
            pltpu.VMEM((_CHUNK,), jnp.int32),
            pltpu.VMEM((_CHUNK,), jnp.float32),
            pltpu.VMEM((_L,), jnp.float32),
            pltpu.VMEM((_L,), jnp.float32),
        ],
    )(mz, charge_i32, slopes16, intercepts16)


def kernel(mz, charge, slopes, intercepts):
    mz = mz.reshape(-1)
    charge_i32 = charge.reshape(-1).astype(jnp.int32)
    slopes16 = jnp.zeros((_L,), jnp.float32).at[:slopes.shape[0]].set(slopes)
    intercepts16 = (
        jnp.zeros((_L,), jnp.float32).at[:intercepts.shape[0]].set(intercepts)
    )
    ccs = _sc_call(mz, charge_i32, slopes16, intercepts16)
    return ccs.reshape(-1, 1)


# speedup vs baseline: 1.0774x; 1.0774x over previous
import functools

import jax
import jax.numpy as jnp
from jax import lax
from jax.experimental import pallas as pl
from jax.experimental.pallas import tpu as pltpu
from jax.experimental.pallas import tpu_sc as plsc

_BATCH = 16384
_NW = 32
_CHUNK = _BATCH // _NW
_L = 16
_MAX_CHARGE = 6


def _sc_body(mz_hbm, ch_hbm, sl_hbm, ic_hbm, out_hbm, mz_v, ch_v, out_v,
             sl_v, ic_v, sem):
    wid = lax.axis_index("s") * 2 + lax.axis_index("c")
    base = wid * _CHUNK
    c1 = pltpu.async_copy(mz_hbm.at[pl.ds(base, _CHUNK)], mz_v, sem)
    c2 = pltpu.async_copy(ch_hbm.at[pl.ds(base, _CHUNK)], ch_v, sem)
    c3 = pltpu.async_copy(sl_hbm, sl_v, sem)
    c4 = pltpu.async_copy(ic_hbm, ic_v, sem)
    c1.wait()
    c2.wait()
    c3.wait()
    c4.wait()

    for i in range(_CHUNK // _L):
        sl_ix = pl.ds(i * _L, _L)
        x = mz_v[sl_ix]
        c = ch_v[sl_ix]
        c = jnp.minimum(jnp.maximum(c, 0), _MAX_CHARGE)
        slope = plsc.load_gather(sl_v, [c])
        inter = plsc.load_gather(ic_v, [c])
        h = x * jnp.float32(0.5)
        yi = jnp.int32(0x5F3759DF) - (plsc.bitcast(x, jnp.int32) >> 1)
        y = plsc.bitcast(yi, jnp.float32)
        y = y * (jnp.float32(1.5) - h * y * y)
        y = y * (jnp.float32(1.5) - h * y * y)
        y = y * (jnp.float32(1.5) - h * y * y)
        s = x * y
        out_v[sl_ix] = slope * s + inter

    pltpu.sync_copy(out_v, out_hbm.at[pl.ds(base, _CHUNK)])


@jax.jit
def _sc_call(mz, charge_i32, slopes16, intercepts16):
    mesh = plsc.VectorSubcoreMesh(core_axis_name="c", subcore_axis_name="s")
    return pl.kernel(
        _sc_body,
        out_type=jax.ShapeDtypeStruct((_BATCH,), jnp.float32),
        mesh=mesh,
        compiler_params=pltpu.CompilerParams(needs_layout_passes=False),
        scratch_types=[
            pltpu.VMEM((_CHUNK,), jnp.float32),
            pltpu.VMEM((_CHUNK,), jnp.int32),
            pltpu.VMEM((_CHUNK,), jnp.float32),
            pltpu.VMEM((_L,), jnp.float32),
            pltpu.VMEM((_L,), jnp.float32),
            pltpu.SemaphoreType.DMA,
        ],
    )(mz, charge_i32, slopes16, intercepts16)


def kernel(mz, charge, slopes, intercepts):
    mz = mz.reshape(-1)
    charge_i32 = charge.reshape(-1).astype(jnp.int32)
    slopes16 = jnp.zeros((_L,), jnp.float32).at[:slopes.shape[0]].set(slopes)
    intercepts16 = (
        jnp.zeros((_L,), jnp.float32).at[:intercepts.shape[0]].set(intercepts)
    )
    ccs = _sc_call(mz, charge_i32, slopes16, intercepts16)
    return ccs.reshape(-1, 1)
